# baseline (speedup 1.0000x reference)
import functools

import jax
import jax.numpy as jnp
from jax.experimental import pallas as pl
from jax.experimental.pallas import tpu as pltpu


def _mlp_stats_body(x_ref, w1_ref, b1_ref, w2_ref, b2_ref,
                    y_ref, s1_ref, s2_ref):
    i = pl.program_id(0)
    x = x_ref[...]
    h = jnp.maximum(
        jnp.dot(x, w1_ref[...], preferred_element_type=jnp.float32)
        + b1_ref[...], 0.0)
    y = (jnp.dot(h, w2_ref[...], preferred_element_type=jnp.float32)
         + b2_ref[...])
    y_ref[...] = y

    @pl.when(i == 0)
    def _init():
        s1_ref[...] = jnp.zeros_like(s1_ref)
        s2_ref[...] = jnp.zeros_like(s2_ref)

    s1_ref[...] += jnp.sum(y, axis=0, keepdims=True)
    s2_ref[...] += jnp.sum(y * y, axis=0, keepdims=True)


def _mlp_stats(x, w1, b1, w2, b2, block):
    n = x.shape[0]
    grid = n // block
    y, s1, s2 = pl.pallas_call(
        _mlp_stats_body,
        grid=(grid,),
        in_specs=[
            pl.BlockSpec((block, 128), lambda i: (i, 0)),
            pl.BlockSpec((128, 128), lambda i: (0, 0)),
            pl.BlockSpec((1, 128), lambda i: (0, 0)),
            pl.BlockSpec((128, 128), lambda i: (0, 0)),
            pl.BlockSpec((1, 128), lambda i: (0, 0)),
        ],
        out_specs=[
            pl.BlockSpec((block, 128), lambda i: (i, 0)),
            pl.BlockSpec((1, 128), lambda i: (0, 0)),
            pl.BlockSpec((1, 128), lambda i: (0, 0)),
        ],
        out_shape=[
            jax.ShapeDtypeStruct((n, 128), jnp.float32),
            jax.ShapeDtypeStruct((1, 128), jnp.float32),
            jax.ShapeDtypeStruct((1, 128), jnp.float32),
        ],
    )(x, w1, b1.reshape(1, 128), w2, b2.reshape(1, 128))
    return y, s1, s2


def _bn_body(y_ref, mu_ref, rstd_ref, gamma_ref, beta_ref, o_ref):
    o_ref[...] = ((y_ref[...] - mu_ref[...]) * rstd_ref[...]
                  * gamma_ref[...] + beta_ref[...])


def _bn_apply(y, mu, rstd, gamma, beta, block):
    n = y.shape[0]
    return pl.pallas_call(
        _bn_body,
        grid=(n // block,),
        in_specs=[
            pl.BlockSpec((block, 128), lambda i: (i, 0)),
            pl.BlockSpec((1, 128), lambda i: (0, 0)),
            pl.BlockSpec((1, 128), lambda i: (0, 0)),
            pl.BlockSpec((1, 128), lambda i: (0, 0)),
            pl.BlockSpec((1, 128), lambda i: (0, 0)),
        ],
        out_specs=pl.BlockSpec((block, 128), lambda i: (i, 0)),
        out_shape=jax.ShapeDtypeStruct((n, 128), jnp.float32),
    )(y, mu, rstd, gamma.reshape(1, 128), beta.reshape(1, 128))


def _mlp_bn(x, w1, b1, w2, b2, gamma, beta, block):
    n = x.shape[0]
    y, s1, s2 = _mlp_stats(x, w1, b1, w2, b2, block)
    mu = s1 / n
    var = s2 / n - mu * mu
    rstd = jax.lax.rsqrt(var + 1e-5)
    return _bn_apply(y, mu, rstd, gamma, beta, block)


def kernel(nh, eh, edge_index,
           nf_W1, nf_b1, nf_W2, nf_b2, nf_eps, nf_gamma, nf_beta,
           ef_W1, ef_b1, ef_W2, ef_b2, ef_eps, ef_gamma, ef_beta):
    N = nh.shape[0]
    src, dst = edge_index[0], edge_index[1]
    src_nh = nh[src]
    dst_nh = nh[dst]
    attn = jnp.sum((src_nh + eh) * dst_nh, axis=-1)
    seg_max = jax.ops.segment_max(attn, dst, num_segments=N)
    seg_max = jnp.where(jnp.isfinite(seg_max), seg_max, 0.0)
    ex = jnp.exp(attn - seg_max[dst])
    denom = jax.ops.segment_sum(ex, dst, num_segments=N)
    a = ex / denom[dst]
    nz = jax.ops.segment_sum(src_nh * a[:, None], dst, num_segments=N)
    n_pre = (1.0 + nf_eps) * nh + nz
    e_pre = (1.0 + ef_eps) * eh + nz[src] - nz[dst]
    n_h = _mlp_bn(n_pre, nf_W1, nf_b1, nf_W2, nf_b2, nf_gamma, nf_beta, 1000)
    e_h = _mlp_bn(e_pre, ef_W1, ef_b1, ef_W2, ef_b2, ef_gamma, ef_beta, 1000)
    return (n_h, e_h)


# speedup vs baseline: 1.0840x; 1.0840x over previous
import functools

import jax
import jax.numpy as jnp
from jax import lax
from jax.experimental import pallas as pl
from jax.experimental.pallas import tpu as pltpu
from jax.experimental.pallas import tpu_sc as plsc

_NC = 2
_NS = 16
_NW = _NC * _NS
_CK = 80


def _epre_body(epsp_hbm, eh_hbm, nz_hbm, src_hbm, dst_hbm, out_hbm,
               epsp_v, eh_v, s_v, d_v, si_v, di_v, sem):
    c = lax.axis_index("c")
    s = lax.axis_index("s")
    wid = s * _NC + c
    E = eh_hbm.shape[0]
    ew = E // _NW
    base = wid * ew
    pltpu.sync_copy(epsp_hbm, epsp_v)
    eps_sl = [epsp_v[pl.ds(t * 16, 16)] for t in range(8)]

    def chunk(i, carry):
        b = base + i * _CK
        pltpu.sync_copy(src_hbm.at[pl.ds(b, _CK)], si_v)
        pltpu.sync_copy(dst_hbm.at[pl.ds(b, _CK)], di_v)
        pltpu.sync_copy(eh_hbm.at[pl.ds(b, _CK)], eh_v)
        pltpu.async_copy(nz_hbm.at[si_v], s_v, sem).wait()
        pltpu.async_copy(nz_hbm.at[di_v], d_v, sem).wait()

        def row(j, carry2):
            for t in range(8):
                sl = pl.ds(t * 16, 16)
                eh_v[j, sl] = (eh_v[j, sl] * eps_sl[t]
                               + s_v[j, sl] - d_v[j, sl])
            return carry2

        lax.fori_loop(0, _CK, row, 0)
        pltpu.sync_copy(eh_v, out_hbm.at[pl.ds(b, _CK)])
        return carry

    lax.fori_loop(0, ew // _CK, chunk, 0)


def _epre_sc(epsp, eh, nz, src, dst):
    E = eh.shape[0]
    mesh = plsc.VectorSubcoreMesh(core_axis_name="c", subcore_axis_name="s")
    f = pl.kernel(
        _epre_body,
        out_type=jax.ShapeDtypeStruct((E, 128), jnp.float32),
        mesh=mesh,
        scratch_types=[
            pltpu.VMEM((128,), jnp.float32),
            pltpu.VMEM((_CK, 128), jnp.float32),
            pltpu.VMEM((_CK, 128), jnp.float32),
            pltpu.VMEM((_CK, 128), jnp.float32),
            pltpu.VMEM((_CK,), jnp.int32),
            pltpu.VMEM((_CK,), jnp.int32),
            pltpu.SemaphoreType.DMA,
        ],
    )
    return f(epsp, eh, nz, src, dst)


def _mlp_stats_body(x_ref, w1_ref, b1_ref, w2_ref, b2_ref,
                    y_ref, s1_ref, s2_ref):
    i = pl.program_id(0)
    x = x_ref[...]
    h = jnp.maximum(
        jnp.dot(x, w1_ref[...], preferred_element_type=jnp.float32)
        + b1_ref[...], 0.0)
    y = (jnp.dot(h, w2_ref[...], preferred_element_type=jnp.float32)
         + b2_ref[...])
    y_ref[...] = y

    @pl.when(i == 0)
    def _init():
        s1_ref[...] = jnp.zeros_like(s1_ref)
        s2_ref[...] = jnp.zeros_like(s2_ref)

    s1_ref[...] += jnp.sum(y, axis=0, keepdims=True)
    s2_ref[...] += jnp.sum(y * y, axis=0, keepdims=True)


def _mlp_stats(x, w1, b1, w2, b2, block):
    n = x.shape[0]
    grid = n // block
    y, s1, s2 = pl.pallas_call(
        _mlp_stats_body,
        grid=(grid,),
        in_specs=[
            pl.BlockSpec((block, 128), lambda i: (i, 0)),
            pl.BlockSpec((128, 128), lambda i: (0, 0)),
            pl.BlockSpec((1, 128), lambda i: (0, 0)),
            pl.BlockSpec((128, 128), lambda i: (0, 0)),
            pl.BlockSpec((1, 128), lambda i: (0, 0)),
        ],
        out_specs=[
            pl.BlockSpec((block, 128), lambda i: (i, 0)),
            pl.BlockSpec((1, 128), lambda i: (0, 0)),
            pl.BlockSpec((1, 128), lambda i: (0, 0)),
        ],
        out_shape=[
            jax.ShapeDtypeStruct((n, 128), jnp.float32),
            jax.ShapeDtypeStruct((1, 128), jnp.float32),
            jax.ShapeDtypeStruct((1, 128), jnp.float32),
        ],
    )(x, w1, b1.reshape(1, 128), w2, b2.reshape(1, 128))
    return y, s1, s2


def _bn_body(y_ref, mu_ref, rstd_ref, gamma_ref, beta_ref, o_ref):
    o_ref[...] = ((y_ref[...] - mu_ref[...]) * rstd_ref[...]
                  * gamma_ref[...] + beta_ref[...])


def _bn_apply(y, mu, rstd, gamma, beta, block):
    n = y.shape[0]
    return pl.pallas_call(
        _bn_body,
        grid=(n // block,),
        in_specs=[
            pl.BlockSpec((block, 128), lambda i: (i, 0)),
            pl.BlockSpec((1, 128), lambda i: (0, 0)),
            pl.BlockSpec((1, 128), lambda i: (0, 0)),
            pl.BlockSpec((1, 128), lambda i: (0, 0)),
            pl.BlockSpec((1, 128), lambda i: (0, 0)),
        ],
        out_specs=pl.BlockSpec((block, 128), lambda i: (i, 0)),
        out_shape=jax.ShapeDtypeStruct((n, 128), jnp.float32),
    )(y, mu, rstd, gamma.reshape(1, 128), beta.reshape(1, 128))


def _mlp_bn(x, w1, b1, w2, b2, gamma, beta, block):
    n = x.shape[0]
    y, s1, s2 = _mlp_stats(x, w1, b1, w2, b2, block)
    mu = s1 / n
    var = s2 / n - mu * mu
    rstd = jax.lax.rsqrt(var + 1e-5)
    return _bn_apply(y, mu, rstd, gamma, beta, block)


def kernel(nh, eh, edge_index,
           nf_W1, nf_b1, nf_W2, nf_b2, nf_eps, nf_gamma, nf_beta,
           ef_W1, ef_b1, ef_W2, ef_b2, ef_eps, ef_gamma, ef_beta):
    N = nh.shape[0]
    src, dst = edge_index[0], edge_index[1]
    src_nh = nh[src]
    dst_nh = nh[dst]
    attn = jnp.sum((src_nh + eh) * dst_nh, axis=-1)
    seg_max = jax.ops.segment_max(attn, dst, num_segments=N)
    seg_max = jnp.where(jnp.isfinite(seg_max), seg_max, 0.0)
    ex = jnp.exp(attn - seg_max[dst])
    denom = jax.ops.segment_sum(ex, dst, num_segments=N)
    a = ex / denom[dst]
    nz = jax.ops.segment_sum(src_nh * a[:, None], dst, num_segments=N)
    n_pre = (1.0 + nf_eps) * nh + nz
    e_pre = _epre_sc(1.0 + ef_eps, eh, nz, src, dst)
    n_h = _mlp_bn(n_pre, nf_W1, nf_b1, nf_W2, nf_b2, nf_gamma, nf_beta, 1000)
    e_h = _mlp_bn(e_pre, ef_W1, ef_b1, ef_W2, ef_b2, ef_gamma, ef_beta, 1000)
    return (n_h, e_h)
